# baseline (speedup 1.0000x reference)
import functools

import jax
import jax.numpy as jnp
from jax import lax
from jax.experimental import pallas as pl
from jax.experimental.pallas import tpu as pltpu
from jax.experimental.pallas import tpu_sc as plsc

N = 50000
V = 50000
S = 16
EPS = 1e-5

_NC, _NS = 2, 16
_NW = _NC * _NS

_PC = 8
_CHI = _PC * S
_NCH = N // _PC
_WBASE = 195
_WLAST = _NCH - (_NW - 1) * _WBASE
_IDXSPAN = _WLAST * _CHI
_STW = 576

_BP = 1000
_NB = N // _BP
_BPR = 2000


def _sc_pool_call(t1, t2, t3, i1, i2, i3, x1, x2, x3):
    mesh = plsc.VectorSubcoreMesh(core_axis_name="c", subcore_axis_name="s")

    @functools.partial(
        pl.kernel,
        mesh=mesh,
        out_type=[jax.ShapeDtypeStruct((N, 64), jnp.float32)] * 3 +
                 [jax.ShapeDtypeStruct((_NW, _STW), jnp.float32)],
        compiler_params=pltpu.CompilerParams(use_tc_tiling_on_sc=False),
        scratch_types=[
            pltpu.VMEM((_IDXSPAN,), jnp.int32),
            pltpu.VMEM((_CHI, 64), jnp.float32),
            pltpu.VMEM((_CHI, 64), jnp.float32),
            pltpu.VMEM((_PC, 64), jnp.float32),
            pltpu.VMEM((_PC, 64), jnp.float32),
            pltpu.VMEM((_PC, 64), jnp.float32),
            pltpu.VMEM((_PC, 64), jnp.float32),
            pltpu.VMEM((_STW,), jnp.float32),
            pltpu.SemaphoreType.DMA,
            pltpu.SemaphoreType.DMA,
            pltpu.SemaphoreType.DMA,
            pltpu.SemaphoreType.DMA,
        ],
    )
    def body(t1h, t2h, t3h, i1h, i2h, i3h, x1h, x2h, x3h,
             m1h, m2h, m3h, sth,
             idx_v, rows_a, rows_b, pxb_a, pxb_b, mh_a, mh_b, stats_v,
             sga, sgb, swa, swb):
        wid = lax.axis_index("s") * _NC + lax.axis_index("c")
        start = wid * _WBASE
        nch = jnp.where(wid == _NW - 1, _WLAST, _WBASE)
        npair = nch // 2

        for sidx, (th, ih, xh, mh) in enumerate(
                ((t1h, i1h, x1h, m1h), (t2h, i2h, x2h, m2h),
                 (t3h, i3h, x3h, m3h))):
            pltpu.sync_copy(ih.at[pl.ds(start * _CHI, _IDXSPAN)], idx_v)

            def fire(li, rows, pxb, sem):
                pltpu.async_copy(
                    th.at[idx_v.at[pl.ds(li * _CHI, _CHI)]], rows, sem)
                pltpu.async_copy(xh.at[pl.ds((start + li) * _PC, _PC)],
                                 pxb, sem)

            def wait_fire(li, rows, pxb, sem):
                pltpu.make_async_copy(
                    th.at[idx_v.at[pl.ds(li * _CHI, _CHI)]], rows,
                    sem).wait()
                pltpu.make_async_copy(
                    xh.at[pl.ds((start + li) * _PC, _PC)], pxb, sem).wait()

            def drain_wb(mh_v, sem):
                pltpu.make_async_copy(mh_v, mh.at[pl.ds(0, _PC)],
                                      sem).wait()

            def compute(li, rows, pxb, mh_v, wsem, carry):
                sm = list(carry[0:4])
                sq = list(carry[4:8])
                cx = list(carry[8:12])
                drain_wb(mh_v, wsem)
                for p in range(_PC):
                    for g in range(4):
                        sl = pl.ds(g * 16, 16)
                        pxv = pxb[p, sl]
                        v = rows[p * S, sl]
                        mx = v
                        smp = v
                        sq[g] = sq[g] + v * v
                        for t in range(1, S):
                            v = rows[p * S + t, sl]
                            mx = jnp.maximum(mx, v)
                            smp = smp + v
                            sq[g] = sq[g] + v * v
                        mh_v[p, sl] = mx - pxv
                        sm[g] = sm[g] + smp
                        cx[g] = cx[g] + pxv * smp
                pltpu.async_copy(mh_v,
                                 mh.at[pl.ds((start + li) * _PC, _PC)],
                                 wsem)
                return tuple(sm) + tuple(sq) + tuple(cx)

            fire(0, rows_a, pxb_a, sga)
            pltpu.async_copy(mh_a, mh.at[pl.ds(start * _PC, _PC)], swa)
            pltpu.async_copy(mh_b, mh.at[pl.ds((start + 1) * _PC, _PC)],
                             swb)

            zeros = jnp.zeros((16,), jnp.float32)
            carry0 = (zeros,) * 12

            def pair(ip, carry):
                ia = 2 * ip
                ib = ia + 1
                fire(ib, rows_b, pxb_b, sgb)
                wait_fire(ia, rows_a, pxb_a, sga)
                carry = compute(ia, rows_a, pxb_a, mh_a, swa, carry)
                fire(ia + 2, rows_a, pxb_a, sga)
                wait_fire(ib, rows_b, pxb_b, sgb)
                carry = compute(ib, rows_b, pxb_b, mh_b, swb, carry)
                return carry

            carry = lax.fori_loop(0, npair, pair, carry0)

            wait_fire(nch - 1, rows_a, pxb_a, sga)
            carry = compute(nch - 1, rows_a, pxb_a, mh_a, swa, carry)
            drain_wb(mh_a, swa)
            drain_wb(mh_b, swb)

            for g in range(4):
                stats_v[pl.ds(sidx * 192 + g * 16, 16)] = carry[g]
                stats_v[pl.ds(sidx * 192 + 64 + g * 16, 16)] = carry[4 + g]
                stats_v[pl.ds(sidx * 192 + 128 + g * 16, 16)] = \
                    carry[8 + g]

        pltpu.sync_copy(stats_v, sth.at[wid])

    return body(t1, t2, t3, i1, i2, i3, x1, x2, x3)


def _bn_affine(s_ref, q_ref, count):
    m = s_ref[...] * (1.0 / count)
    v = q_ref[...] * (1.0 / count) - m * m
    return m, lax.rsqrt(v + EPS)


def _prep_body(vx1, vf1, vx2, vf2, vx3, vf3, pxyz,
               w31, wf1, w32, wf2, w33, wf3, wcat,
               vp1, vp2, vp3, ypos, px1, px2, px3, ys, yq,
               xs1, xq1, xs2, xq2, xs3, xq3):
    f32 = jnp.float32
    vp1[...] = jnp.dot(vx1[...], w31[...], preferred_element_type=f32) + \
               jnp.dot(vf1[...], wf1[...], preferred_element_type=f32)
    vp2[...] = jnp.dot(vx2[...], w32[...], preferred_element_type=f32) + \
               jnp.dot(vf2[...], wf2[...], preferred_element_type=f32)
    vp3[...] = jnp.dot(vx3[...], w33[...], preferred_element_type=f32) + \
               jnp.dot(vf3[...], wf3[...], preferred_element_type=f32)
    p = jnp.dot(pxyz[...], wcat[...], preferred_element_type=f32)
    yp = p[:, 0:128]
    ypos[...] = yp
    a1 = p[:, 128:192]
    a2 = p[:, 192:256]
    a3 = p[:, 256:320]
    px1[...] = a1
    px2[...] = a2
    px3[...] = a3

    @pl.when(pl.program_id(0) == 0)
    def _():
        for r in (ys, yq, xs1, xq1, xs2, xq2, xs3, xq3):
            r[...] = jnp.zeros_like(r)

    ys[...] += jnp.sum(yp, axis=0, keepdims=True)
    yq[...] += jnp.sum(yp * yp, axis=0, keepdims=True)
    xs1[...] += jnp.sum(a1, axis=0, keepdims=True)
    xq1[...] += jnp.sum(a1 * a1, axis=0, keepdims=True)
    xs2[...] += jnp.sum(a2, axis=0, keepdims=True)
    xq2[...] += jnp.sum(a2 * a2, axis=0, keepdims=True)
    xs3[...] += jnp.sum(a3, axis=0, keepdims=True)
    xq3[...] += jnp.sum(a3 * a3, axis=0, keepdims=True)


def _prep_call(vx1, vf1, vx2, vf2, vx3, vf3, pxyz,
               w31, wf1, w32, wf2, w33, wf3, wcat):
    f32 = jnp.float32
    blk = lambda c: pl.BlockSpec((_BPR, c), lambda i: (i, 0))
    full = lambda r, c: pl.BlockSpec((r, c), lambda i: (0, 0))
    return pl.pallas_call(
        _prep_body,
        grid=(N // _BPR,),
        in_specs=[blk(3), blk(32), blk(3), blk(64), blk(3), blk(64),
                  blk(3),
                  full(3, 64), full(32, 64), full(3, 64), full(64, 64),
                  full(3, 64), full(64, 64), full(3, 320)],
        out_specs=[blk(64), blk(64), blk(64), blk(128),
                   blk(64), blk(64), blk(64),
                   full(1, 128), full(1, 128)] + [full(1, 64)] * 6,
        out_shape=[jax.ShapeDtypeStruct((V, 64), f32)] * 3 +
                  [jax.ShapeDtypeStruct((N, 128), f32)] +
                  [jax.ShapeDtypeStruct((N, 64), f32)] * 3 +
                  [jax.ShapeDtypeStruct((1, 128), f32),
                   jax.ShapeDtypeStruct((1, 128), f32)] +
                  [jax.ShapeDtypeStruct((1, 64), f32)] * 6,
        compiler_params=pltpu.CompilerParams(
            dimension_semantics=("arbitrary",)),
    )(vx1, vf1, vx2, vf2, vx3, vf3, pxyz,
      w31, wf1, w32, wf2, w33, wf3, wcat)


def _head1_body(m1, m2, m3, stats, xs1, xq1, xs2, xq2, xs3, xq3,
                wr1, wr2, wr3, yraw, ys, yq):
    cnt = float(N * S)
    st = jnp.sum(stats[...], axis=0, keepdims=True)
    f32 = jnp.float32
    ps = []
    for sidx, (m, xs, xq) in enumerate(((m1, xs1, xq1), (m2, xs2, xq2),
                                        (m3, xs3, xq3))):
        sv = st[:, sidx * 192:sidx * 192 + 64]
        qv = st[:, sidx * 192 + 64:sidx * 192 + 128]
        cv = st[:, sidx * 192 + 128:sidx * 192 + 192]
        hs = sv - float(S) * xs[...]
        hq = qv - 2.0 * cv + float(S) * xq[...]
        mu = hs * (1.0 / cnt)
        var = hq * (1.0 / cnt) - mu * mu
        rs = lax.rsqrt(var + EPS)
        ps.append(jnp.maximum((m[...] - mu) * rs, 0.0))
    y = jnp.dot(ps[0], wr1[...], preferred_element_type=f32) + \
        jnp.dot(ps[1], wr2[...], preferred_element_type=f32) + \
        jnp.dot(ps[2], wr3[...], preferred_element_type=f32)
    yraw[...] = y

    @pl.when(pl.program_id(0) == 0)
    def _():
        ys[...] = jnp.zeros_like(ys)
        yq[...] = jnp.zeros_like(yq)

    ys[...] += jnp.sum(y, axis=0, keepdims=True)
    yq[...] += jnp.sum(y * y, axis=0, keepdims=True)


def _head1_call(m1, m2, m3, stats, pxstats, wr1, wr2, wr3):
    f32 = jnp.float32
    blk64 = pl.BlockSpec((_BP, 64), lambda i: (i, 0))
    st64 = pl.BlockSpec((1, 64), lambda i: (0, 0))
    stw = pl.BlockSpec((_NW, _STW), lambda i: (0, 0))
    w = pl.BlockSpec((64, 128), lambda i: (0, 0))
    return pl.pallas_call(
        _head1_body,
        grid=(_NB,),
        in_specs=[blk64, blk64, blk64, stw] + [st64] * 6 + [w, w, w],
        out_specs=[pl.BlockSpec((_BP, 128), lambda i: (i, 0)),
                   pl.BlockSpec((1, 128), lambda i: (0, 0)),
                   pl.BlockSpec((1, 128), lambda i: (0, 0))],
        out_shape=[jax.ShapeDtypeStruct((N, 128), f32),
                   jax.ShapeDtypeStruct((1, 128), f32),
                   jax.ShapeDtypeStruct((1, 128), f32)],
        compiler_params=pltpu.CompilerParams(
            dimension_semantics=("arbitrary",)),
    )(m1, m2, m3, stats, *pxstats, wr1, wr2, wr3)


def _head2_body(yraw, ypos, rs_, rq_, ps_, pq_, wfg1, wct1,
                z1, z2, s1, q1, s2, q2):
    cnt = float(N)
    mur, rsr = _bn_affine(rs_, rq_, cnt)
    mup, rsp = _bn_affine(ps_, pq_, cnt)
    feat = jnp.maximum((yraw[...] - mur) * rsr + (ypos[...] - mup) * rsp,
                       0.0)
    f32 = jnp.float32
    a = jnp.dot(feat, wfg1[...], preferred_element_type=f32)
    b = jnp.dot(feat, wct1[...], preferred_element_type=f32)
    z1[...] = a
    z2[...] = b

    @pl.when(pl.program_id(0) == 0)
    def _():
        s1[...] = jnp.zeros_like(s1)
        q1[...] = jnp.zeros_like(q1)
        s2[...] = jnp.zeros_like(s2)
        q2[...] = jnp.zeros_like(q2)

    s1[...] += jnp.sum(a, axis=0, keepdims=True)
    q1[...] += jnp.sum(a * a, axis=0, keepdims=True)
    s2[...] += jnp.sum(b, axis=0, keepdims=True)
    q2[...] += jnp.sum(b * b, axis=0, keepdims=True)


def _head2_call(yraw, ypos, rs_, rq_, ps_, pq_, wfg1, wct1):
    f32 = jnp.float32
    blk128 = pl.BlockSpec((_BP, 128), lambda i: (i, 0))
    st = pl.BlockSpec((1, 128), lambda i: (0, 0))
    w = pl.BlockSpec((128, 64), lambda i: (0, 0))
    st64 = pl.BlockSpec((1, 64), lambda i: (0, 0))
    return pl.pallas_call(
        _head2_body,
        grid=(_NB,),
        in_specs=[blk128, blk128, st, st, st, st, w, w],
        out_specs=[pl.BlockSpec((_BP, 64), lambda i: (i, 0))] * 2 +
                  [st64, st64, st64, st64],
        out_shape=[jax.ShapeDtypeStruct((N, 64), f32)] * 2 +
                  [jax.ShapeDtypeStruct((1, 64), f32)] * 4,
        compiler_params=pltpu.CompilerParams(
            dimension_semantics=("arbitrary",)),
    )(yraw, ypos, rs_, rq_, ps_, pq_, wfg1, wct1)


def _head3_body(z1, z2, s1, q1, s2, q2, wfg2, wct2, bcat, out):
    cnt = float(N)
    mu1, rs1 = _bn_affine(s1, q1, cnt)
    mu2, rs2 = _bn_affine(s2, q2, cnt)
    a1 = jnp.maximum((z1[...] - mu1) * rs1, 0.0)
    a2 = jnp.maximum((z2[...] - mu2) * rs2, 0.0)
    f32 = jnp.float32
    out[...] = jnp.dot(a1, wfg2[...], preferred_element_type=f32) + \
               jnp.dot(a2, wct2[...], preferred_element_type=f32) + \
               bcat[...]


def _head3_call(z1, z2, s1, q1, s2, q2, wfg2, wct2, bcat):
    f32 = jnp.float32
    blk64 = pl.BlockSpec((_BP, 64), lambda i: (i, 0))
    st = pl.BlockSpec((1, 64), lambda i: (0, 0))
    w = pl.BlockSpec((64, 6), lambda i: (0, 0))
    return pl.pallas_call(
        _head3_body,
        grid=(_NB,),
        in_specs=[blk64, blk64, st, st, st, st, w, w,
                  pl.BlockSpec((1, 6), lambda i: (0, 0))],
        out_specs=pl.BlockSpec((_BP, 6), lambda i: (i, 0)),
        out_shape=jax.ShapeDtypeStruct((N, 6), f32),
        compiler_params=pltpu.CompilerParams(
            dimension_semantics=("arbitrary",)),
    )(z1, z2, s1, q1, s2, q2, wfg2, wct2, bcat)


def kernel(point_xyz, vx1, vx2, vx3, vf1, vf2, vf3, idx1, idx2, idx3,
           W_s1, W_s2, W_s3, W_raw, W_pos, W_fg1, W_fg2, b_fg,
           W_ct1, W_ct2, b_ct):
    f32 = jnp.float32
    w31, wf1 = W_s1[:3], W_s1[3:]
    w32, wf2 = W_s2[:3], W_s2[3:]
    w33, wf3 = W_s3[:3], W_s3[3:]
    wcat = jnp.concatenate([W_pos, w31, w32, w33], axis=1)
    wr1, wr2, wr3 = W_raw[0:64], W_raw[64:128], W_raw[128:192]
    wfg2p = jnp.concatenate([W_fg2, jnp.zeros((64, 3), f32)], axis=1)
    wct2p = jnp.concatenate([jnp.zeros((64, 3), f32), W_ct2], axis=1)
    bcat = jnp.concatenate([b_fg, b_ct]).reshape(1, 6)

    (vp1, vp2, vp3, ypos, px1, px2, px3, ps_, pq_,
     xs1, xq1, xs2, xq2, xs3, xq3) = _prep_call(
        vx1, vf1, vx2, vf2, vx3, vf3, point_xyz,
        w31, wf1, w32, wf2, w33, wf3, wcat)

    m1, m2, m3, stats = _sc_pool_call(
        vp1, vp2, vp3, idx1.reshape(-1), idx2.reshape(-1),
        idx3.reshape(-1), px1, px2, px3)

    yraw, rs_, rq_ = _head1_call(m1, m2, m3, stats,
                                 (xs1, xq1, xs2, xq2, xs3, xq3),
                                 wr1, wr2, wr3)
    z1, z2, zs1, zq1, zs2, zq2 = _head2_call(yraw, ypos, rs_, rq_,
                                             ps_, pq_, W_fg1, W_ct1)
    return _head3_call(z1, z2, zs1, zq1, zs2, zq2, wfg2p, wct2p, bcat)


# speedup vs baseline: 1.1196x; 1.1196x over previous
import functools

import jax
import jax.numpy as jnp
from jax import lax
from jax.experimental import pallas as pl
from jax.experimental.pallas import tpu as pltpu
from jax.experimental.pallas import tpu_sc as plsc

N = 50000
V = 50000
S = 16
EPS = 1e-5

_NC, _NS = 2, 16
_NW = _NC * _NS

_PC = 4
_CHI = _PC * S
_NCH = N // _PC
_WBASE = 389
_WLAST = _NCH - (_NW - 1) * _WBASE
_IDXSPAN = _WLAST * _CHI
_STW = 576

_BP = 1000
_NB = N // _BP
_BPR = 2000


def _sc_pool_call(t1, t2, t3, i1, i2, i3, x1, x2, x3):
    mesh = plsc.VectorSubcoreMesh(core_axis_name="c", subcore_axis_name="s")

    @functools.partial(
        pl.kernel,
        mesh=mesh,
        out_type=[jax.ShapeDtypeStruct((N, 64), jnp.float32)] * 3 +
                 [jax.ShapeDtypeStruct((_NW, _STW), jnp.float32)],
        compiler_params=pltpu.CompilerParams(use_tc_tiling_on_sc=False),
        scratch_types=[
            pltpu.VMEM((_IDXSPAN,), jnp.int32),
            pltpu.VMEM((_CHI, 64), jnp.float32),
            pltpu.VMEM((_CHI, 64), jnp.float32),
            pltpu.VMEM((_PC, 64), jnp.float32),
            pltpu.VMEM((_PC, 64), jnp.float32),
            pltpu.VMEM((_PC, 64), jnp.float32),
            pltpu.VMEM((_PC, 64), jnp.float32),
            pltpu.VMEM((_STW,), jnp.float32),
            pltpu.SemaphoreType.DMA,
            pltpu.SemaphoreType.DMA,
            pltpu.SemaphoreType.DMA,
            pltpu.SemaphoreType.DMA,
        ],
    )
    def body(t1h, t2h, t3h, i1h, i2h, i3h, x1h, x2h, x3h,
             m1h, m2h, m3h, sth,
             idx_v, rows_a, rows_b, pxb_a, pxb_b, mh_a, mh_b, stats_v,
             sga, sgb, swa, swb):
        wid = lax.axis_index("s") * _NC + lax.axis_index("c")
        start = wid * _WBASE
        nch = jnp.where(wid == _NW - 1, _WLAST, _WBASE)
        npair = nch // 2

        for sidx, (th, ih, xh, mh) in enumerate(
                ((t1h, i1h, x1h, m1h), (t2h, i2h, x2h, m2h),
                 (t3h, i3h, x3h, m3h))):
            pltpu.sync_copy(ih.at[pl.ds(start * _CHI, _IDXSPAN)], idx_v)

            def fire(li, rows, pxb, sem):
                pltpu.async_copy(
                    th.at[idx_v.at[pl.ds(li * _CHI, _CHI)]], rows, sem)
                pltpu.async_copy(xh.at[pl.ds((start + li) * _PC, _PC)],
                                 pxb, sem)

            def wait_fire(li, rows, pxb, sem):
                pltpu.make_async_copy(
                    th.at[idx_v.at[pl.ds(li * _CHI, _CHI)]], rows,
                    sem).wait()
                pltpu.make_async_copy(
                    xh.at[pl.ds((start + li) * _PC, _PC)], pxb, sem).wait()

            def drain_wb(mh_v, sem):
                pltpu.make_async_copy(mh_v, mh.at[pl.ds(0, _PC)],
                                      sem).wait()

            def compute(li, rows, pxb, mh_v, wsem, carry):
                sm = list(carry[0:4])
                sq = list(carry[4:8])
                cx = list(carry[8:12])
                drain_wb(mh_v, wsem)
                for p in range(_PC):
                    for g in range(4):
                        sl = pl.ds(g * 16, 16)
                        pxv = pxb[p, sl]
                        v = rows[p * S, sl]
                        mx = v
                        smp = v
                        sq[g] = sq[g] + v * v
                        for t in range(1, S):
                            v = rows[p * S + t, sl]
                            mx = jnp.maximum(mx, v)
                            smp = smp + v
                            sq[g] = sq[g] + v * v
                        mh_v[p, sl] = mx - pxv
                        sm[g] = sm[g] + smp
                        cx[g] = cx[g] + pxv * smp
                pltpu.async_copy(mh_v,
                                 mh.at[pl.ds((start + li) * _PC, _PC)],
                                 wsem)
                return tuple(sm) + tuple(sq) + tuple(cx)

            fire(0, rows_a, pxb_a, sga)
            pltpu.async_copy(mh_a, mh.at[pl.ds(start * _PC, _PC)], swa)
            pltpu.async_copy(mh_b, mh.at[pl.ds((start + 1) * _PC, _PC)],
                             swb)

            zeros = jnp.zeros((16,), jnp.float32)
            carry0 = (zeros,) * 12

            def pair(ip, carry):
                ia = 2 * ip
                ib = ia + 1
                fire(ib, rows_b, pxb_b, sgb)
                wait_fire(ia, rows_a, pxb_a, sga)
                carry = compute(ia, rows_a, pxb_a, mh_a, swa, carry)
                fire(ia + 2, rows_a, pxb_a, sga)
                wait_fire(ib, rows_b, pxb_b, sgb)
                carry = compute(ib, rows_b, pxb_b, mh_b, swb, carry)
                return carry

            carry = lax.fori_loop(0, npair, pair, carry0)

            wait_fire(nch - 1, rows_a, pxb_a, sga)
            carry = compute(nch - 1, rows_a, pxb_a, mh_a, swa, carry)
            drain_wb(mh_a, swa)
            drain_wb(mh_b, swb)

            for g in range(4):
                stats_v[pl.ds(sidx * 192 + g * 16, 16)] = carry[g]
                stats_v[pl.ds(sidx * 192 + 64 + g * 16, 16)] = carry[4 + g]
                stats_v[pl.ds(sidx * 192 + 128 + g * 16, 16)] = \
                    carry[8 + g]

        pltpu.sync_copy(stats_v, sth.at[wid])

    return body(t1, t2, t3, i1, i2, i3, x1, x2, x3)


def _bn_affine(s_ref, q_ref, count):
    m = s_ref[...] * (1.0 / count)
    v = q_ref[...] * (1.0 / count) - m * m
    return m, lax.rsqrt(v + EPS)


def _prep_body(vx1, vf1, vx2, vf2, vx3, vf3, pxyz,
               w31, wf1, w32, wf2, w33, wf3, wcat,
               vp1, vp2, vp3, ypos, px1, px2, px3, ys, yq,
               xs1, xq1, xs2, xq2, xs3, xq3):
    f32 = jnp.float32
    vp1[...] = jnp.dot(vx1[...], w31[...], preferred_element_type=f32) + \
               jnp.dot(vf1[...], wf1[...], preferred_element_type=f32)
    vp2[...] = jnp.dot(vx2[...], w32[...], preferred_element_type=f32) + \
               jnp.dot(vf2[...], wf2[...], preferred_element_type=f32)
    vp3[...] = jnp.dot(vx3[...], w33[...], preferred_element_type=f32) + \
               jnp.dot(vf3[...], wf3[...], preferred_element_type=f32)
    p = jnp.dot(pxyz[...], wcat[...], preferred_element_type=f32)
    yp = p[:, 0:128]
    ypos[...] = yp
    a1 = p[:, 128:192]
    a2 = p[:, 192:256]
    a3 = p[:, 256:320]
    px1[...] = a1
    px2[...] = a2
    px3[...] = a3

    @pl.when(pl.program_id(0) == 0)
    def _():
        for r in (ys, yq, xs1, xq1, xs2, xq2, xs3, xq3):
            r[...] = jnp.zeros_like(r)

    ys[...] += jnp.sum(yp, axis=0, keepdims=True)
    yq[...] += jnp.sum(yp * yp, axis=0, keepdims=True)
    xs1[...] += jnp.sum(a1, axis=0, keepdims=True)
    xq1[...] += jnp.sum(a1 * a1, axis=0, keepdims=True)
    xs2[...] += jnp.sum(a2, axis=0, keepdims=True)
    xq2[...] += jnp.sum(a2 * a2, axis=0, keepdims=True)
    xs3[...] += jnp.sum(a3, axis=0, keepdims=True)
    xq3[...] += jnp.sum(a3 * a3, axis=0, keepdims=True)


def _prep_call(vx1, vf1, vx2, vf2, vx3, vf3, pxyz,
               w31, wf1, w32, wf2, w33, wf3, wcat):
    f32 = jnp.float32
    blk = lambda c: pl.BlockSpec((_BPR, c), lambda i: (i, 0))
    full = lambda r, c: pl.BlockSpec((r, c), lambda i: (0, 0))
    return pl.pallas_call(
        _prep_body,
        grid=(N // _BPR,),
        in_specs=[blk(3), blk(32), blk(3), blk(64), blk(3), blk(64),
                  blk(3),
                  full(3, 64), full(32, 64), full(3, 64), full(64, 64),
                  full(3, 64), full(64, 64), full(3, 320)],
        out_specs=[blk(64), blk(64), blk(64), blk(128),
                   blk(64), blk(64), blk(64),
                   full(1, 128), full(1, 128)] + [full(1, 64)] * 6,
        out_shape=[jax.ShapeDtypeStruct((V, 64), f32)] * 3 +
                  [jax.ShapeDtypeStruct((N, 128), f32)] +
                  [jax.ShapeDtypeStruct((N, 64), f32)] * 3 +
                  [jax.ShapeDtypeStruct((1, 128), f32),
                   jax.ShapeDtypeStruct((1, 128), f32)] +
                  [jax.ShapeDtypeStruct((1, 64), f32)] * 6,
        compiler_params=pltpu.CompilerParams(
            dimension_semantics=("arbitrary",)),
    )(vx1, vf1, vx2, vf2, vx3, vf3, pxyz,
      w31, wf1, w32, wf2, w33, wf3, wcat)


def _head1_body(m1, m2, m3, stats, xs1, xq1, xs2, xq2, xs3, xq3,
                wr1, wr2, wr3, yraw, ys, yq):
    cnt = float(N * S)
    st = jnp.sum(stats[...], axis=0, keepdims=True)
    f32 = jnp.float32
    ps = []
    for sidx, (m, xs, xq) in enumerate(((m1, xs1, xq1), (m2, xs2, xq2),
                                        (m3, xs3, xq3))):
        sv = st[:, sidx * 192:sidx * 192 + 64]
        qv = st[:, sidx * 192 + 64:sidx * 192 + 128]
        cv = st[:, sidx * 192 + 128:sidx * 192 + 192]
        hs = sv - float(S) * xs[...]
        hq = qv - 2.0 * cv + float(S) * xq[...]
        mu = hs * (1.0 / cnt)
        var = hq * (1.0 / cnt) - mu * mu
        rs = lax.rsqrt(var + EPS)
        ps.append(jnp.maximum((m[...] - mu) * rs, 0.0))
    y = jnp.dot(ps[0], wr1[...], preferred_element_type=f32) + \
        jnp.dot(ps[1], wr2[...], preferred_element_type=f32) + \
        jnp.dot(ps[2], wr3[...], preferred_element_type=f32)
    yraw[...] = y

    @pl.when(pl.program_id(0) == 0)
    def _():
        ys[...] = jnp.zeros_like(ys)
        yq[...] = jnp.zeros_like(yq)

    ys[...] += jnp.sum(y, axis=0, keepdims=True)
    yq[...] += jnp.sum(y * y, axis=0, keepdims=True)


def _head1_call(m1, m2, m3, stats, pxstats, wr1, wr2, wr3):
    f32 = jnp.float32
    blk64 = pl.BlockSpec((_BP, 64), lambda i: (i, 0))
    st64 = pl.BlockSpec((1, 64), lambda i: (0, 0))
    stw = pl.BlockSpec((_NW, _STW), lambda i: (0, 0))
    w = pl.BlockSpec((64, 128), lambda i: (0, 0))
    return pl.pallas_call(
        _head1_body,
        grid=(_NB,),
        in_specs=[blk64, blk64, blk64, stw] + [st64] * 6 + [w, w, w],
        out_specs=[pl.BlockSpec((_BP, 128), lambda i: (i, 0)),
                   pl.BlockSpec((1, 128), lambda i: (0, 0)),
                   pl.BlockSpec((1, 128), lambda i: (0, 0))],
        out_shape=[jax.ShapeDtypeStruct((N, 128), f32),
                   jax.ShapeDtypeStruct((1, 128), f32),
                   jax.ShapeDtypeStruct((1, 128), f32)],
        compiler_params=pltpu.CompilerParams(
            dimension_semantics=("arbitrary",)),
    )(m1, m2, m3, stats, *pxstats, wr1, wr2, wr3)


def _head2_body(yraw, ypos, rs_, rq_, ps_, pq_, wfg1, wct1,
                z1, z2, s1, q1, s2, q2):
    cnt = float(N)
    mur, rsr = _bn_affine(rs_, rq_, cnt)
    mup, rsp = _bn_affine(ps_, pq_, cnt)
    feat = jnp.maximum((yraw[...] - mur) * rsr + (ypos[...] - mup) * rsp,
                       0.0)
    f32 = jnp.float32
    a = jnp.dot(feat, wfg1[...], preferred_element_type=f32)
    b = jnp.dot(feat, wct1[...], preferred_element_type=f32)
    z1[...] = a
    z2[...] = b

    @pl.when(pl.program_id(0) == 0)
    def _():
        s1[...] = jnp.zeros_like(s1)
        q1[...] = jnp.zeros_like(q1)
        s2[...] = jnp.zeros_like(s2)
        q2[...] = jnp.zeros_like(q2)

    s1[...] += jnp.sum(a, axis=0, keepdims=True)
    q1[...] += jnp.sum(a * a, axis=0, keepdims=True)
    s2[...] += jnp.sum(b, axis=0, keepdims=True)
    q2[...] += jnp.sum(b * b, axis=0, keepdims=True)


def _head2_call(yraw, ypos, rs_, rq_, ps_, pq_, wfg1, wct1):
    f32 = jnp.float32
    blk128 = pl.BlockSpec((_BP, 128), lambda i: (i, 0))
    st = pl.BlockSpec((1, 128), lambda i: (0, 0))
    w = pl.BlockSpec((128, 64), lambda i: (0, 0))
    st64 = pl.BlockSpec((1, 64), lambda i: (0, 0))
    return pl.pallas_call(
        _head2_body,
        grid=(_NB,),
        in_specs=[blk128, blk128, st, st, st, st, w, w],
        out_specs=[pl.BlockSpec((_BP, 64), lambda i: (i, 0))] * 2 +
                  [st64, st64, st64, st64],
        out_shape=[jax.ShapeDtypeStruct((N, 64), f32)] * 2 +
                  [jax.ShapeDtypeStruct((1, 64), f32)] * 4,
        compiler_params=pltpu.CompilerParams(
            dimension_semantics=("arbitrary",)),
    )(yraw, ypos, rs_, rq_, ps_, pq_, wfg1, wct1)


def _head3_body(z1, z2, s1, q1, s2, q2, wfg2, wct2, bcat, out):
    cnt = float(N)
    mu1, rs1 = _bn_affine(s1, q1, cnt)
    mu2, rs2 = _bn_affine(s2, q2, cnt)
    a1 = jnp.maximum((z1[...] - mu1) * rs1, 0.0)
    a2 = jnp.maximum((z2[...] - mu2) * rs2, 0.0)
    f32 = jnp.float32
    out[...] = jnp.dot(a1, wfg2[...], preferred_element_type=f32) + \
               jnp.dot(a2, wct2[...], preferred_element_type=f32) + \
               bcat[...]


def _head3_call(z1, z2, s1, q1, s2, q2, wfg2, wct2, bcat):
    f32 = jnp.float32
    blk64 = pl.BlockSpec((_BP, 64), lambda i: (i, 0))
    st = pl.BlockSpec((1, 64), lambda i: (0, 0))
    w = pl.BlockSpec((64, 6), lambda i: (0, 0))
    return pl.pallas_call(
        _head3_body,
        grid=(_NB,),
        in_specs=[blk64, blk64, st, st, st, st, w, w,
                  pl.BlockSpec((1, 6), lambda i: (0, 0))],
        out_specs=pl.BlockSpec((_BP, 6), lambda i: (i, 0)),
        out_shape=jax.ShapeDtypeStruct((N, 6), f32),
        compiler_params=pltpu.CompilerParams(
            dimension_semantics=("arbitrary",)),
    )(z1, z2, s1, q1, s2, q2, wfg2, wct2, bcat)


def kernel(point_xyz, vx1, vx2, vx3, vf1, vf2, vf3, idx1, idx2, idx3,
           W_s1, W_s2, W_s3, W_raw, W_pos, W_fg1, W_fg2, b_fg,
           W_ct1, W_ct2, b_ct):
    f32 = jnp.float32
    w31, wf1 = W_s1[:3], W_s1[3:]
    w32, wf2 = W_s2[:3], W_s2[3:]
    w33, wf3 = W_s3[:3], W_s3[3:]
    wcat = jnp.concatenate([W_pos, w31, w32, w33], axis=1)
    wr1, wr2, wr3 = W_raw[0:64], W_raw[64:128], W_raw[128:192]
    wfg2p = jnp.concatenate([W_fg2, jnp.zeros((64, 3), f32)], axis=1)
    wct2p = jnp.concatenate([jnp.zeros((64, 3), f32), W_ct2], axis=1)
    bcat = jnp.concatenate([b_fg, b_ct]).reshape(1, 6)

    (vp1, vp2, vp3, ypos, px1, px2, px3, ps_, pq_,
     xs1, xq1, xs2, xq2, xs3, xq3) = _prep_call(
        vx1, vf1, vx2, vf2, vx3, vf3, point_xyz,
        w31, wf1, w32, wf2, w33, wf3, wcat)

    m1, m2, m3, stats = _sc_pool_call(
        vp1, vp2, vp3, idx1.reshape(-1), idx2.reshape(-1),
        idx3.reshape(-1), px1, px2, px3)

    yraw, rs_, rq_ = _head1_call(m1, m2, m3, stats,
                                 (xs1, xq1, xs2, xq2, xs3, xq3),
                                 wr1, wr2, wr3)
    z1, z2, zs1, zq1, zs2, zq2 = _head2_call(yraw, ypos, rs_, rq_,
                                             ps_, pq_, W_fg1, W_ct1)
    return _head3_call(z1, z2, zs1, zq1, zs2, zq2, wfg2p, wct2p, bcat)


# speedup vs baseline: 1.1611x; 1.0371x over previous
import functools

import jax
import jax.numpy as jnp
from jax import lax
from jax.experimental import pallas as pl
from jax.experimental.pallas import tpu as pltpu
from jax.experimental.pallas import tpu_sc as plsc

N = 50000
V = 50000
S = 16
EPS = 1e-5

_NC, _NS = 2, 16
_NW = _NC * _NS

_PC = 4
_CHI = _PC * S
_NCH = N // _PC
_WSMALL = 389
_WBIG = 391
_IDXSPAN = _WBIG * _CHI
_STW = 576

_BP = 1000
_NB = N // _BP
_BPR = 2000


def _sc_pool_call(t1, t2, t3, i1, i2, i3, x1, x2, x3):
    mesh = plsc.VectorSubcoreMesh(core_axis_name="c", subcore_axis_name="s")

    @functools.partial(
        pl.kernel,
        mesh=mesh,
        out_type=[jax.ShapeDtypeStruct((N, 64), jnp.float32)] * 3 +
                 [jax.ShapeDtypeStruct((_NW, _STW), jnp.float32)],
        compiler_params=pltpu.CompilerParams(use_tc_tiling_on_sc=False),
        scratch_types=[
            pltpu.VMEM((_IDXSPAN,), jnp.int32),
            pltpu.VMEM((_CHI, 64), jnp.float32),
            pltpu.VMEM((_CHI, 64), jnp.float32),
            pltpu.VMEM((_PC, 64), jnp.float32),
            pltpu.VMEM((_PC, 64), jnp.float32),
            pltpu.VMEM((_PC, 64), jnp.float32),
            pltpu.VMEM((_PC, 64), jnp.float32),
            pltpu.VMEM((_STW,), jnp.float32),
            pltpu.SemaphoreType.DMA,
            pltpu.SemaphoreType.DMA,
            pltpu.SemaphoreType.DMA,
            pltpu.SemaphoreType.DMA,
        ],
    )
    def body(t1h, t2h, t3h, i1h, i2h, i3h, x1h, x2h, x3h,
             m1h, m2h, m3h, sth,
             idx_v, rows_a, rows_b, pxb_a, pxb_b, mh_a, mh_b, stats_v,
             sga, sgb, swa, swb):
        wid = lax.axis_index("s") * _NC + lax.axis_index("c")
        start = wid * _WSMALL + 2 * jnp.maximum(wid - 6, 0)
        nch = jnp.where(wid < 6, _WSMALL, _WBIG)
        npair = nch // 2

        for sidx, (th, ih, xh, mh) in enumerate(
                ((t1h, i1h, x1h, m1h), (t2h, i2h, x2h, m2h),
                 (t3h, i3h, x3h, m3h))):
            pltpu.sync_copy(ih.at[pl.ds(start * _CHI, _IDXSPAN)], idx_v)

            def fire(li, rows, pxb, sem):
                pltpu.async_copy(
                    th.at[idx_v.at[pl.ds(li * _CHI, _CHI)]], rows, sem)
                pltpu.async_copy(xh.at[pl.ds((start + li) * _PC, _PC)],
                                 pxb, sem)

            def wait_fire(li, rows, pxb, sem):
                pltpu.make_async_copy(
                    th.at[idx_v.at[pl.ds(li * _CHI, _CHI)]], rows,
                    sem).wait()
                pltpu.make_async_copy(
                    xh.at[pl.ds((start + li) * _PC, _PC)], pxb, sem).wait()

            def drain_wb(mh_v, sem):
                pltpu.make_async_copy(mh_v, mh.at[pl.ds(0, _PC)],
                                      sem).wait()

            def compute(li, rows, pxb, mh_v, wsem, carry):
                sm = list(carry[0:4])
                sq = list(carry[4:8])
                cx = list(carry[8:12])
                drain_wb(mh_v, wsem)
                for p in range(_PC):
                    for g in range(4):
                        sl = pl.ds(g * 16, 16)
                        pxv = pxb[p, sl]
                        v = rows[p * S, sl]
                        mx = v
                        smp = v
                        sq[g] = sq[g] + v * v
                        for t in range(1, S):
                            v = rows[p * S + t, sl]
                            mx = jnp.maximum(mx, v)
                            smp = smp + v
                            sq[g] = sq[g] + v * v
                        mh_v[p, sl] = mx - pxv
                        sm[g] = sm[g] + smp
                        cx[g] = cx[g] + pxv * smp
                pltpu.async_copy(mh_v,
                                 mh.at[pl.ds((start + li) * _PC, _PC)],
                                 wsem)
                return tuple(sm) + tuple(sq) + tuple(cx)

            fire(0, rows_a, pxb_a, sga)
            pltpu.async_copy(mh_a, mh.at[pl.ds(start * _PC, _PC)], swa)
            pltpu.async_copy(mh_b, mh.at[pl.ds((start + 1) * _PC, _PC)],
                             swb)

            zeros = jnp.zeros((16,), jnp.float32)
            carry0 = (zeros,) * 12

            def pair(ip, carry):
                ia = 2 * ip
                ib = ia + 1
                fire(ib, rows_b, pxb_b, sgb)
                wait_fire(ia, rows_a, pxb_a, sga)
                carry = compute(ia, rows_a, pxb_a, mh_a, swa, carry)
                fire(ia + 2, rows_a, pxb_a, sga)
                wait_fire(ib, rows_b, pxb_b, sgb)
                carry = compute(ib, rows_b, pxb_b, mh_b, swb, carry)
                return carry

            carry = lax.fori_loop(0, npair, pair, carry0)

            wait_fire(nch - 1, rows_a, pxb_a, sga)
            carry = compute(nch - 1, rows_a, pxb_a, mh_a, swa, carry)
            drain_wb(mh_a, swa)
            drain_wb(mh_b, swb)

            for g in range(4):
                stats_v[pl.ds(sidx * 192 + g * 16, 16)] = carry[g]
                stats_v[pl.ds(sidx * 192 + 64 + g * 16, 16)] = carry[4 + g]
                stats_v[pl.ds(sidx * 192 + 128 + g * 16, 16)] = \
                    carry[8 + g]

        pltpu.sync_copy(stats_v, sth.at[wid])

    return body(t1, t2, t3, i1, i2, i3, x1, x2, x3)


def _bn_affine(s_ref, q_ref, count):
    m = s_ref[...] * (1.0 / count)
    v = q_ref[...] * (1.0 / count) - m * m
    return m, lax.rsqrt(v + EPS)


def _prep_body(vx1, vf1, vx2, vf2, vx3, vf3, pxyz,
               w31, wf1, w32, wf2, w33, wf3, wcat,
               vp1, vp2, vp3, ypos, px1, px2, px3, ys, yq,
               xs1, xq1, xs2, xq2, xs3, xq3):
    f32 = jnp.float32
    vp1[...] = jnp.dot(vx1[...], w31[...], preferred_element_type=f32) + \
               jnp.dot(vf1[...], wf1[...], preferred_element_type=f32)
    vp2[...] = jnp.dot(vx2[...], w32[...], preferred_element_type=f32) + \
               jnp.dot(vf2[...], wf2[...], preferred_element_type=f32)
    vp3[...] = jnp.dot(vx3[...], w33[...], preferred_element_type=f32) + \
               jnp.dot(vf3[...], wf3[...], preferred_element_type=f32)
    p = jnp.dot(pxyz[...], wcat[...], preferred_element_type=f32)
    yp = p[:, 0:128]
    ypos[...] = yp
    a1 = p[:, 128:192]
    a2 = p[:, 192:256]
    a3 = p[:, 256:320]
    px1[...] = a1
    px2[...] = a2
    px3[...] = a3

    @pl.when(pl.program_id(0) == 0)
    def _():
        for r in (ys, yq, xs1, xq1, xs2, xq2, xs3, xq3):
            r[...] = jnp.zeros_like(r)

    ys[...] += jnp.sum(yp, axis=0, keepdims=True)
    yq[...] += jnp.sum(yp * yp, axis=0, keepdims=True)
    xs1[...] += jnp.sum(a1, axis=0, keepdims=True)
    xq1[...] += jnp.sum(a1 * a1, axis=0, keepdims=True)
    xs2[...] += jnp.sum(a2, axis=0, keepdims=True)
    xq2[...] += jnp.sum(a2 * a2, axis=0, keepdims=True)
    xs3[...] += jnp.sum(a3, axis=0, keepdims=True)
    xq3[...] += jnp.sum(a3 * a3, axis=0, keepdims=True)


def _prep_call(vx1, vf1, vx2, vf2, vx3, vf3, pxyz,
               w31, wf1, w32, wf2, w33, wf3, wcat):
    f32 = jnp.float32
    blk = lambda c: pl.BlockSpec((_BPR, c), lambda i: (i, 0))
    full = lambda r, c: pl.BlockSpec((r, c), lambda i: (0, 0))
    return pl.pallas_call(
        _prep_body,
        grid=(N // _BPR,),
        in_specs=[blk(3), blk(32), blk(3), blk(64), blk(3), blk(64),
                  blk(3),
                  full(3, 64), full(32, 64), full(3, 64), full(64, 64),
                  full(3, 64), full(64, 64), full(3, 320)],
        out_specs=[blk(64), blk(64), blk(64), blk(128),
                   blk(64), blk(64), blk(64),
                   full(1, 128), full(1, 128)] + [full(1, 64)] * 6,
        out_shape=[jax.ShapeDtypeStruct((V, 64), f32)] * 3 +
                  [jax.ShapeDtypeStruct((N, 128), f32)] +
                  [jax.ShapeDtypeStruct((N, 64), f32)] * 3 +
                  [jax.ShapeDtypeStruct((1, 128), f32),
                   jax.ShapeDtypeStruct((1, 128), f32)] +
                  [jax.ShapeDtypeStruct((1, 64), f32)] * 6,
        compiler_params=pltpu.CompilerParams(
            dimension_semantics=("arbitrary",)),
    )(vx1, vf1, vx2, vf2, vx3, vf3, pxyz,
      w31, wf1, w32, wf2, w33, wf3, wcat)


def _head1_body(m1, m2, m3, stats, xs1, xq1, xs2, xq2, xs3, xq3,
                wr1, wr2, wr3, yraw, ys, yq):
    cnt = float(N * S)
    st = jnp.sum(stats[...], axis=0, keepdims=True)
    f32 = jnp.float32
    ps = []
    for sidx, (m, xs, xq) in enumerate(((m1, xs1, xq1), (m2, xs2, xq2),
                                        (m3, xs3, xq3))):
        sv = st[:, sidx * 192:sidx * 192 + 64]
        qv = st[:, sidx * 192 + 64:sidx * 192 + 128]
        cv = st[:, sidx * 192 + 128:sidx * 192 + 192]
        hs = sv - float(S) * xs[...]
        hq = qv - 2.0 * cv + float(S) * xq[...]
        mu = hs * (1.0 / cnt)
        var = hq * (1.0 / cnt) - mu * mu
        rs = lax.rsqrt(var + EPS)
        ps.append(jnp.maximum((m[...] - mu) * rs, 0.0))
    y = jnp.dot(ps[0], wr1[...], preferred_element_type=f32) + \
        jnp.dot(ps[1], wr2[...], preferred_element_type=f32) + \
        jnp.dot(ps[2], wr3[...], preferred_element_type=f32)
    yraw[...] = y

    @pl.when(pl.program_id(0) == 0)
    def _():
        ys[...] = jnp.zeros_like(ys)
        yq[...] = jnp.zeros_like(yq)

    ys[...] += jnp.sum(y, axis=0, keepdims=True)
    yq[...] += jnp.sum(y * y, axis=0, keepdims=True)


def _head1_call(m1, m2, m3, stats, pxstats, wr1, wr2, wr3):
    f32 = jnp.float32
    blk64 = pl.BlockSpec((_BP, 64), lambda i: (i, 0))
    st64 = pl.BlockSpec((1, 64), lambda i: (0, 0))
    stw = pl.BlockSpec((_NW, _STW), lambda i: (0, 0))
    w = pl.BlockSpec((64, 128), lambda i: (0, 0))
    return pl.pallas_call(
        _head1_body,
        grid=(_NB,),
        in_specs=[blk64, blk64, blk64, stw] + [st64] * 6 + [w, w, w],
        out_specs=[pl.BlockSpec((_BP, 128), lambda i: (i, 0)),
                   pl.BlockSpec((1, 128), lambda i: (0, 0)),
                   pl.BlockSpec((1, 128), lambda i: (0, 0))],
        out_shape=[jax.ShapeDtypeStruct((N, 128), f32),
                   jax.ShapeDtypeStruct((1, 128), f32),
                   jax.ShapeDtypeStruct((1, 128), f32)],
        compiler_params=pltpu.CompilerParams(
            dimension_semantics=("arbitrary",)),
    )(m1, m2, m3, stats, *pxstats, wr1, wr2, wr3)


def _head2_body(yraw, ypos, rs_, rq_, ps_, pq_, wfg1, wct1,
                z1, z2, s1, q1, s2, q2):
    cnt = float(N)
    mur, rsr = _bn_affine(rs_, rq_, cnt)
    mup, rsp = _bn_affine(ps_, pq_, cnt)
    feat = jnp.maximum((yraw[...] - mur) * rsr + (ypos[...] - mup) * rsp,
                       0.0)
    f32 = jnp.float32
    a = jnp.dot(feat, wfg1[...], preferred_element_type=f32)
    b = jnp.dot(feat, wct1[...], preferred_element_type=f32)
    z1[...] = a
    z2[...] = b

    @pl.when(pl.program_id(0) == 0)
    def _():
        s1[...] = jnp.zeros_like(s1)
        q1[...] = jnp.zeros_like(q1)
        s2[...] = jnp.zeros_like(s2)
        q2[...] = jnp.zeros_like(q2)

    s1[...] += jnp.sum(a, axis=0, keepdims=True)
    q1[...] += jnp.sum(a * a, axis=0, keepdims=True)
    s2[...] += jnp.sum(b, axis=0, keepdims=True)
    q2[...] += jnp.sum(b * b, axis=0, keepdims=True)


def _head2_call(yraw, ypos, rs_, rq_, ps_, pq_, wfg1, wct1):
    f32 = jnp.float32
    blk128 = pl.BlockSpec((_BP, 128), lambda i: (i, 0))
    st = pl.BlockSpec((1, 128), lambda i: (0, 0))
    w = pl.BlockSpec((128, 64), lambda i: (0, 0))
    st64 = pl.BlockSpec((1, 64), lambda i: (0, 0))
    return pl.pallas_call(
        _head2_body,
        grid=(_NB,),
        in_specs=[blk128, blk128, st, st, st, st, w, w],
        out_specs=[pl.BlockSpec((_BP, 64), lambda i: (i, 0))] * 2 +
                  [st64, st64, st64, st64],
        out_shape=[jax.ShapeDtypeStruct((N, 64), f32)] * 2 +
                  [jax.ShapeDtypeStruct((1, 64), f32)] * 4,
        compiler_params=pltpu.CompilerParams(
            dimension_semantics=("arbitrary",)),
    )(yraw, ypos, rs_, rq_, ps_, pq_, wfg1, wct1)


def _head3_body(z1, z2, s1, q1, s2, q2, wfg2, wct2, bcat, out):
    cnt = float(N)
    mu1, rs1 = _bn_affine(s1, q1, cnt)
    mu2, rs2 = _bn_affine(s2, q2, cnt)
    a1 = jnp.maximum((z1[...] - mu1) * rs1, 0.0)
    a2 = jnp.maximum((z2[...] - mu2) * rs2, 0.0)
    f32 = jnp.float32
    out[...] = jnp.dot(a1, wfg2[...], preferred_element_type=f32) + \
               jnp.dot(a2, wct2[...], preferred_element_type=f32) + \
               bcat[...]


def _head3_call(z1, z2, s1, q1, s2, q2, wfg2, wct2, bcat):
    f32 = jnp.float32
    blk64 = pl.BlockSpec((_BP, 64), lambda i: (i, 0))
    st = pl.BlockSpec((1, 64), lambda i: (0, 0))
    w = pl.BlockSpec((64, 6), lambda i: (0, 0))
    return pl.pallas_call(
        _head3_body,
        grid=(_NB,),
        in_specs=[blk64, blk64, st, st, st, st, w, w,
                  pl.BlockSpec((1, 6), lambda i: (0, 0))],
        out_specs=pl.BlockSpec((_BP, 6), lambda i: (i, 0)),
        out_shape=jax.ShapeDtypeStruct((N, 6), f32),
        compiler_params=pltpu.CompilerParams(
            dimension_semantics=("arbitrary",)),
    )(z1, z2, s1, q1, s2, q2, wfg2, wct2, bcat)


def kernel(point_xyz, vx1, vx2, vx3, vf1, vf2, vf3, idx1, idx2, idx3,
           W_s1, W_s2, W_s3, W_raw, W_pos, W_fg1, W_fg2, b_fg,
           W_ct1, W_ct2, b_ct):
    f32 = jnp.float32
    w31, wf1 = W_s1[:3], W_s1[3:]
    w32, wf2 = W_s2[:3], W_s2[3:]
    w33, wf3 = W_s3[:3], W_s3[3:]
    wcat = jnp.concatenate([W_pos, w31, w32, w33], axis=1)
    wr1, wr2, wr3 = W_raw[0:64], W_raw[64:128], W_raw[128:192]
    wfg2p = jnp.concatenate([W_fg2, jnp.zeros((64, 3), f32)], axis=1)
    wct2p = jnp.concatenate([jnp.zeros((64, 3), f32), W_ct2], axis=1)
    bcat = jnp.concatenate([b_fg, b_ct]).reshape(1, 6)

    (vp1, vp2, vp3, ypos, px1, px2, px3, ps_, pq_,
     xs1, xq1, xs2, xq2, xs3, xq3) = _prep_call(
        vx1, vf1, vx2, vf2, vx3, vf3, point_xyz,
        w31, wf1, w32, wf2, w33, wf3, wcat)

    m1, m2, m3, stats = _sc_pool_call(
        vp1, vp2, vp3, idx1.reshape(-1), idx2.reshape(-1),
        idx3.reshape(-1), px1, px2, px3)

    yraw, rs_, rq_ = _head1_call(m1, m2, m3, stats,
                                 (xs1, xq1, xs2, xq2, xs3, xq3),
                                 wr1, wr2, wr3)
    z1, z2, zs1, zq1, zs2, zq2 = _head2_call(yraw, ypos, rs_, rq_,
                                             ps_, pq_, W_fg1, W_ct1)
    return _head3_call(z1, z2, zs1, zq1, zs2, zq2, wfg2p, wct2p, bcat)


# speedup vs baseline: 1.2155x; 1.0468x over previous
import functools

import jax
import jax.numpy as jnp
from jax import lax
from jax.experimental import pallas as pl
from jax.experimental.pallas import tpu as pltpu
from jax.experimental.pallas import tpu_sc as plsc

N = 50000
V = 50000
S = 16
EPS = 1e-5

_NC, _NS = 2, 16
_NW = _NC * _NS

_PC = 4
_CHI = _PC * S
_NCH = N // _PC
_WSMALL = 389
_WBIG = 391
_IDXSPAN = _WBIG * _CHI
_STW = 576

_BP = 2000
_NB = N // _BP
_BPR = 2000


def _sc_pool_call(t1, t2, t3, i1, i2, i3, x1, x2, x3):
    mesh = plsc.VectorSubcoreMesh(core_axis_name="c", subcore_axis_name="s")

    @functools.partial(
        pl.kernel,
        mesh=mesh,
        out_type=[jax.ShapeDtypeStruct((N, 64), jnp.float32)] * 3 +
                 [jax.ShapeDtypeStruct((_NW, _STW), jnp.float32)],
        compiler_params=pltpu.CompilerParams(use_tc_tiling_on_sc=False),
        scratch_types=[
            pltpu.VMEM((_IDXSPAN,), jnp.int32),
            pltpu.VMEM((_CHI, 64), jnp.float32),
            pltpu.VMEM((_CHI, 64), jnp.float32),
            pltpu.VMEM((_PC, 64), jnp.float32),
            pltpu.VMEM((_PC, 64), jnp.float32),
            pltpu.VMEM((_PC, 64), jnp.float32),
            pltpu.VMEM((_PC, 64), jnp.float32),
            pltpu.VMEM((_STW,), jnp.float32),
            pltpu.SemaphoreType.DMA,
            pltpu.SemaphoreType.DMA,
            pltpu.SemaphoreType.DMA,
            pltpu.SemaphoreType.DMA,
        ],
    )
    def body(t1h, t2h, t3h, i1h, i2h, i3h, x1h, x2h, x3h,
             m1h, m2h, m3h, sth,
             idx_v, rows_a, rows_b, pxb_a, pxb_b, mh_a, mh_b, stats_v,
             sga, sgb, swa, swb):
        wid = lax.axis_index("s") * _NC + lax.axis_index("c")
        start = wid * _WSMALL + 2 * jnp.maximum(wid - 6, 0)
        nch = jnp.where(wid < 6, _WSMALL, _WBIG)
        npair = nch // 2

        for sidx, (th, ih, xh, mh) in enumerate(
                ((t1h, i1h, x1h, m1h), (t2h, i2h, x2h, m2h),
                 (t3h, i3h, x3h, m3h))):
            pltpu.sync_copy(ih.at[pl.ds(start * _CHI, _IDXSPAN)], idx_v)

            def fire(li, rows, pxb, sem):
                pltpu.async_copy(
                    th.at[idx_v.at[pl.ds(li * _CHI, _CHI)]], rows, sem)
                pltpu.async_copy(xh.at[pl.ds((start + li) * _PC, _PC)],
                                 pxb, sem)

            def wait_fire(li, rows, pxb, sem):
                pltpu.make_async_copy(
                    th.at[idx_v.at[pl.ds(li * _CHI, _CHI)]], rows,
                    sem).wait()
                pltpu.make_async_copy(
                    xh.at[pl.ds((start + li) * _PC, _PC)], pxb, sem).wait()

            def drain_wb(mh_v, sem):
                pltpu.make_async_copy(mh_v, mh.at[pl.ds(0, _PC)],
                                      sem).wait()

            def compute(li, rows, pxb, mh_v, wsem, carry):
                sm = list(carry[0:4])
                sq = list(carry[4:8])
                cx = list(carry[8:12])
                drain_wb(mh_v, wsem)
                for p in range(_PC):
                    for g in range(4):
                        sl = pl.ds(g * 16, 16)
                        pxv = pxb[p, sl]
                        v = rows[p * S, sl]
                        mx = v
                        smp = v
                        sq[g] = sq[g] + v * v
                        for t in range(1, S):
                            v = rows[p * S + t, sl]
                            mx = jnp.maximum(mx, v)
                            smp = smp + v
                            sq[g] = sq[g] + v * v
                        mh_v[p, sl] = mx - pxv
                        sm[g] = sm[g] + smp
                        cx[g] = cx[g] + pxv * smp
                pltpu.async_copy(mh_v,
                                 mh.at[pl.ds((start + li) * _PC, _PC)],
                                 wsem)
                return tuple(sm) + tuple(sq) + tuple(cx)

            fire(0, rows_a, pxb_a, sga)
            pltpu.async_copy(mh_a, mh.at[pl.ds(start * _PC, _PC)], swa)
            pltpu.async_copy(mh_b, mh.at[pl.ds((start + 1) * _PC, _PC)],
                             swb)

            zeros = jnp.zeros((16,), jnp.float32)
            carry0 = (zeros,) * 12

            def pair(ip, carry):
                ia = 2 * ip
                ib = ia + 1
                fire(ib, rows_b, pxb_b, sgb)
                wait_fire(ia, rows_a, pxb_a, sga)
                carry = compute(ia, rows_a, pxb_a, mh_a, swa, carry)
                fire(ia + 2, rows_a, pxb_a, sga)
                wait_fire(ib, rows_b, pxb_b, sgb)
                carry = compute(ib, rows_b, pxb_b, mh_b, swb, carry)
                return carry

            carry = lax.fori_loop(0, npair, pair, carry0)

            wait_fire(nch - 1, rows_a, pxb_a, sga)
            carry = compute(nch - 1, rows_a, pxb_a, mh_a, swa, carry)
            drain_wb(mh_a, swa)
            drain_wb(mh_b, swb)

            for g in range(4):
                stats_v[pl.ds(sidx * 192 + g * 16, 16)] = carry[g]
                stats_v[pl.ds(sidx * 192 + 64 + g * 16, 16)] = carry[4 + g]
                stats_v[pl.ds(sidx * 192 + 128 + g * 16, 16)] = \
                    carry[8 + g]

        pltpu.sync_copy(stats_v, sth.at[wid])

    return body(t1, t2, t3, i1, i2, i3, x1, x2, x3)


def _bn_affine(s_ref, q_ref, count):
    m = s_ref[...] * (1.0 / count)
    v = q_ref[...] * (1.0 / count) - m * m
    return m, lax.rsqrt(v + EPS)


def _prep_body(vx1, vf1, vx2, vf2, vx3, vf3, pxyz,
               w31, wf1, w32, wf2, w33, wf3, wcat,
               vp1, vp2, vp3, ypos, px1, px2, px3, ys, yq,
               xs1, xq1, xs2, xq2, xs3, xq3):
    f32 = jnp.float32
    vp1[...] = jnp.dot(vx1[...], w31[...], preferred_element_type=f32) + \
               jnp.dot(vf1[...], wf1[...], preferred_element_type=f32)
    vp2[...] = jnp.dot(vx2[...], w32[...], preferred_element_type=f32) + \
               jnp.dot(vf2[...], wf2[...], preferred_element_type=f32)
    vp3[...] = jnp.dot(vx3[...], w33[...], preferred_element_type=f32) + \
               jnp.dot(vf3[...], wf3[...], preferred_element_type=f32)
    p = jnp.dot(pxyz[...], wcat[...], preferred_element_type=f32)
    yp = p[:, 0:128]
    ypos[...] = yp
    a1 = p[:, 128:192]
    a2 = p[:, 192:256]
    a3 = p[:, 256:320]
    px1[...] = a1
    px2[...] = a2
    px3[...] = a3

    @pl.when(pl.program_id(0) == 0)
    def _():
        for r in (ys, yq, xs1, xq1, xs2, xq2, xs3, xq3):
            r[...] = jnp.zeros_like(r)

    ys[...] += jnp.sum(yp, axis=0, keepdims=True)
    yq[...] += jnp.sum(yp * yp, axis=0, keepdims=True)
    xs1[...] += jnp.sum(a1, axis=0, keepdims=True)
    xq1[...] += jnp.sum(a1 * a1, axis=0, keepdims=True)
    xs2[...] += jnp.sum(a2, axis=0, keepdims=True)
    xq2[...] += jnp.sum(a2 * a2, axis=0, keepdims=True)
    xs3[...] += jnp.sum(a3, axis=0, keepdims=True)
    xq3[...] += jnp.sum(a3 * a3, axis=0, keepdims=True)


def _prep_call(vx1, vf1, vx2, vf2, vx3, vf3, pxyz,
               w31, wf1, w32, wf2, w33, wf3, wcat):
    f32 = jnp.float32
    blk = lambda c: pl.BlockSpec((_BPR, c), lambda i: (i, 0))
    full = lambda r, c: pl.BlockSpec((r, c), lambda i: (0, 0))
    return pl.pallas_call(
        _prep_body,
        grid=(N // _BPR,),
        in_specs=[blk(3), blk(32), blk(3), blk(64), blk(3), blk(64),
                  blk(3),
                  full(3, 64), full(32, 64), full(3, 64), full(64, 64),
                  full(3, 64), full(64, 64), full(3, 320)],
        out_specs=[blk(64), blk(64), blk(64), blk(128),
                   blk(64), blk(64), blk(64),
                   full(1, 128), full(1, 128)] + [full(1, 64)] * 6,
        out_shape=[jax.ShapeDtypeStruct((V, 64), f32)] * 3 +
                  [jax.ShapeDtypeStruct((N, 128), f32)] +
                  [jax.ShapeDtypeStruct((N, 64), f32)] * 3 +
                  [jax.ShapeDtypeStruct((1, 128), f32),
                   jax.ShapeDtypeStruct((1, 128), f32)] +
                  [jax.ShapeDtypeStruct((1, 64), f32)] * 6,
        compiler_params=pltpu.CompilerParams(
            dimension_semantics=("arbitrary",)),
    )(vx1, vf1, vx2, vf2, vx3, vf3, pxyz,
      w31, wf1, w32, wf2, w33, wf3, wcat)


def _head1_body(m1, m2, m3, stats, xs1, xq1, xs2, xq2, xs3, xq3,
                wr1, wr2, wr3, yraw, ys, yq):
    cnt = float(N * S)
    st = jnp.sum(stats[...], axis=0, keepdims=True)
    f32 = jnp.float32
    ps = []
    for sidx, (m, xs, xq) in enumerate(((m1, xs1, xq1), (m2, xs2, xq2),
                                        (m3, xs3, xq3))):
        sv = st[:, sidx * 192:sidx * 192 + 64]
        qv = st[:, sidx * 192 + 64:sidx * 192 + 128]
        cv = st[:, sidx * 192 + 128:sidx * 192 + 192]
        hs = sv - float(S) * xs[...]
        hq = qv - 2.0 * cv + float(S) * xq[...]
        mu = hs * (1.0 / cnt)
        var = hq * (1.0 / cnt) - mu * mu
        rs = lax.rsqrt(var + EPS)
        ps.append(jnp.maximum((m[...] - mu) * rs, 0.0))
    y = jnp.dot(ps[0], wr1[...], preferred_element_type=f32) + \
        jnp.dot(ps[1], wr2[...], preferred_element_type=f32) + \
        jnp.dot(ps[2], wr3[...], preferred_element_type=f32)
    yraw[...] = y

    @pl.when(pl.program_id(0) == 0)
    def _():
        ys[...] = jnp.zeros_like(ys)
        yq[...] = jnp.zeros_like(yq)

    ys[...] += jnp.sum(y, axis=0, keepdims=True)
    yq[...] += jnp.sum(y * y, axis=0, keepdims=True)


def _head1_call(m1, m2, m3, stats, pxstats, wr1, wr2, wr3):
    f32 = jnp.float32
    blk64 = pl.BlockSpec((_BP, 64), lambda i: (i, 0))
    st64 = pl.BlockSpec((1, 64), lambda i: (0, 0))
    stw = pl.BlockSpec((_NW, _STW), lambda i: (0, 0))
    w = pl.BlockSpec((64, 128), lambda i: (0, 0))
    return pl.pallas_call(
        _head1_body,
        grid=(_NB,),
        in_specs=[blk64, blk64, blk64, stw] + [st64] * 6 + [w, w, w],
        out_specs=[pl.BlockSpec((_BP, 128), lambda i: (i, 0)),
                   pl.BlockSpec((1, 128), lambda i: (0, 0)),
                   pl.BlockSpec((1, 128), lambda i: (0, 0))],
        out_shape=[jax.ShapeDtypeStruct((N, 128), f32),
                   jax.ShapeDtypeStruct((1, 128), f32),
                   jax.ShapeDtypeStruct((1, 128), f32)],
        compiler_params=pltpu.CompilerParams(
            dimension_semantics=("arbitrary",)),
    )(m1, m2, m3, stats, *pxstats, wr1, wr2, wr3)


def _head2_body(yraw, ypos, rs_, rq_, ps_, pq_, wfg1, wct1,
                z1, z2, s1, q1, s2, q2):
    cnt = float(N)
    mur, rsr = _bn_affine(rs_, rq_, cnt)
    mup, rsp = _bn_affine(ps_, pq_, cnt)
    feat = jnp.maximum((yraw[...] - mur) * rsr + (ypos[...] - mup) * rsp,
                       0.0)
    f32 = jnp.float32
    a = jnp.dot(feat, wfg1[...], preferred_element_type=f32)
    b = jnp.dot(feat, wct1[...], preferred_element_type=f32)
    z1[...] = a
    z2[...] = b

    @pl.when(pl.program_id(0) == 0)
    def _():
        s1[...] = jnp.zeros_like(s1)
        q1[...] = jnp.zeros_like(q1)
        s2[...] = jnp.zeros_like(s2)
        q2[...] = jnp.zeros_like(q2)

    s1[...] += jnp.sum(a, axis=0, keepdims=True)
    q1[...] += jnp.sum(a * a, axis=0, keepdims=True)
    s2[...] += jnp.sum(b, axis=0, keepdims=True)
    q2[...] += jnp.sum(b * b, axis=0, keepdims=True)


def _head2_call(yraw, ypos, rs_, rq_, ps_, pq_, wfg1, wct1):
    f32 = jnp.float32
    blk128 = pl.BlockSpec((_BP, 128), lambda i: (i, 0))
    st = pl.BlockSpec((1, 128), lambda i: (0, 0))
    w = pl.BlockSpec((128, 64), lambda i: (0, 0))
    st64 = pl.BlockSpec((1, 64), lambda i: (0, 0))
    return pl.pallas_call(
        _head2_body,
        grid=(_NB,),
        in_specs=[blk128, blk128, st, st, st, st, w, w],
        out_specs=[pl.BlockSpec((_BP, 64), lambda i: (i, 0))] * 2 +
                  [st64, st64, st64, st64],
        out_shape=[jax.ShapeDtypeStruct((N, 64), f32)] * 2 +
                  [jax.ShapeDtypeStruct((1, 64), f32)] * 4,
        compiler_params=pltpu.CompilerParams(
            dimension_semantics=("arbitrary",)),
    )(yraw, ypos, rs_, rq_, ps_, pq_, wfg1, wct1)


def _head3_body(z1, z2, s1, q1, s2, q2, wfg2, wct2, bcat, out):
    cnt = float(N)
    mu1, rs1 = _bn_affine(s1, q1, cnt)
    mu2, rs2 = _bn_affine(s2, q2, cnt)
    a1 = jnp.maximum((z1[...] - mu1) * rs1, 0.0)
    a2 = jnp.maximum((z2[...] - mu2) * rs2, 0.0)
    f32 = jnp.float32
    out[...] = jnp.dot(a1, wfg2[...], preferred_element_type=f32) + \
               jnp.dot(a2, wct2[...], preferred_element_type=f32) + \
               bcat[...]


def _head3_call(z1, z2, s1, q1, s2, q2, wfg2, wct2, bcat):
    f32 = jnp.float32
    blk64 = pl.BlockSpec((_BP, 64), lambda i: (i, 0))
    st = pl.BlockSpec((1, 64), lambda i: (0, 0))
    w = pl.BlockSpec((64, 6), lambda i: (0, 0))
    return pl.pallas_call(
        _head3_body,
        grid=(_NB,),
        in_specs=[blk64, blk64, st, st, st, st, w, w,
                  pl.BlockSpec((1, 6), lambda i: (0, 0))],
        out_specs=pl.BlockSpec((_BP, 6), lambda i: (i, 0)),
        out_shape=jax.ShapeDtypeStruct((N, 6), f32),
        compiler_params=pltpu.CompilerParams(
            dimension_semantics=("arbitrary",)),
    )(z1, z2, s1, q1, s2, q2, wfg2, wct2, bcat)


def kernel(point_xyz, vx1, vx2, vx3, vf1, vf2, vf3, idx1, idx2, idx3,
           W_s1, W_s2, W_s3, W_raw, W_pos, W_fg1, W_fg2, b_fg,
           W_ct1, W_ct2, b_ct):
    f32 = jnp.float32
    w31, wf1 = W_s1[:3], W_s1[3:]
    w32, wf2 = W_s2[:3], W_s2[3:]
    w33, wf3 = W_s3[:3], W_s3[3:]
    wcat = jnp.concatenate([W_pos, w31, w32, w33], axis=1)
    wr1, wr2, wr3 = W_raw[0:64], W_raw[64:128], W_raw[128:192]
    wfg2p = jnp.concatenate([W_fg2, jnp.zeros((64, 3), f32)], axis=1)
    wct2p = jnp.concatenate([jnp.zeros((64, 3), f32), W_ct2], axis=1)
    bcat = jnp.concatenate([b_fg, b_ct]).reshape(1, 6)

    (vp1, vp2, vp3, ypos, px1, px2, px3, ps_, pq_,
     xs1, xq1, xs2, xq2, xs3, xq3) = _prep_call(
        vx1, vf1, vx2, vf2, vx3, vf3, point_xyz,
        w31, wf1, w32, wf2, w33, wf3, wcat)

    m1, m2, m3, stats = _sc_pool_call(
        vp1, vp2, vp3, idx1.reshape(-1), idx2.reshape(-1),
        idx3.reshape(-1), px1, px2, px3)

    yraw, rs_, rq_ = _head1_call(m1, m2, m3, stats,
                                 (xs1, xq1, xs2, xq2, xs3, xq3),
                                 wr1, wr2, wr3)
    z1, z2, zs1, zq1, zs2, zq2 = _head2_call(yraw, ypos, rs_, rq_,
                                             ps_, pq_, W_fg1, W_ct1)
    return _head3_call(z1, z2, zs1, zq1, zs2, zq2, wfg2p, wct2p, bcat)


# speedup vs baseline: 1.2284x; 1.0106x over previous
import functools

import jax
import jax.numpy as jnp
from jax import lax
from jax.experimental import pallas as pl
from jax.experimental.pallas import tpu as pltpu
from jax.experimental.pallas import tpu_sc as plsc

N = 50000
V = 50000
S = 16
EPS = 1e-5

_NC, _NS = 2, 16
_NW = _NC * _NS

_PC = 4
_CHI = _PC * S
_NCH = N // _PC
_WSMALL = 389
_WBIG = 391
_IDXSPAN = _WBIG * _CHI
_STW = 576

_BP = 5000
_NB = N // _BP
_BPR = 2000


def _sc_pool_call(t1, t2, t3, i1, i2, i3, x1, x2, x3):
    mesh = plsc.VectorSubcoreMesh(core_axis_name="c", subcore_axis_name="s")

    @functools.partial(
        pl.kernel,
        mesh=mesh,
        out_type=[jax.ShapeDtypeStruct((N, 64), jnp.float32)] * 3 +
                 [jax.ShapeDtypeStruct((_NW, _STW), jnp.float32)],
        compiler_params=pltpu.CompilerParams(use_tc_tiling_on_sc=False),
        scratch_types=[
            pltpu.VMEM((_IDXSPAN,), jnp.int32),
            pltpu.VMEM((_CHI, 64), jnp.float32),
            pltpu.VMEM((_CHI, 64), jnp.float32),
            pltpu.VMEM((_PC, 64), jnp.float32),
            pltpu.VMEM((_PC, 64), jnp.float32),
            pltpu.VMEM((_PC, 64), jnp.float32),
            pltpu.VMEM((_PC, 64), jnp.float32),
            pltpu.VMEM((_STW,), jnp.float32),
            pltpu.SemaphoreType.DMA,
            pltpu.SemaphoreType.DMA,
            pltpu.SemaphoreType.DMA,
            pltpu.SemaphoreType.DMA,
        ],
    )
    def body(t1h, t2h, t3h, i1h, i2h, i3h, x1h, x2h, x3h,
             m1h, m2h, m3h, sth,
             idx_v, rows_a, rows_b, pxb_a, pxb_b, mh_a, mh_b, stats_v,
             sga, sgb, swa, swb):
        wid = lax.axis_index("s") * _NC + lax.axis_index("c")
        start = wid * _WSMALL + 2 * jnp.maximum(wid - 6, 0)
        nch = jnp.where(wid < 6, _WSMALL, _WBIG)
        npair = nch // 2

        for sidx, (th, ih, xh, mh) in enumerate(
                ((t1h, i1h, x1h, m1h), (t2h, i2h, x2h, m2h),
                 (t3h, i3h, x3h, m3h))):
            pltpu.sync_copy(ih.at[pl.ds(start * _CHI, _IDXSPAN)], idx_v)

            def fire(li, rows, pxb, sem):
                pltpu.async_copy(
                    th.at[idx_v.at[pl.ds(li * _CHI, _CHI)]], rows, sem)
                pltpu.async_copy(xh.at[pl.ds((start + li) * _PC, _PC)],
                                 pxb, sem)

            def wait_fire(li, rows, pxb, sem):
                pltpu.make_async_copy(
                    th.at[idx_v.at[pl.ds(li * _CHI, _CHI)]], rows,
                    sem).wait()
                pltpu.make_async_copy(
                    xh.at[pl.ds((start + li) * _PC, _PC)], pxb, sem).wait()

            def drain_wb(mh_v, sem):
                pltpu.make_async_copy(mh_v, mh.at[pl.ds(0, _PC)],
                                      sem).wait()

            def compute(li, rows, pxb, mh_v, wsem, carry):
                sm = list(carry[0:4])
                sq = list(carry[4:8])
                cx = list(carry[8:12])
                drain_wb(mh_v, wsem)
                for p in range(_PC):
                    for g in range(4):
                        sl = pl.ds(g * 16, 16)
                        pxv = pxb[p, sl]
                        v = rows[p * S, sl]
                        mx = v
                        smp = v
                        sq[g] = sq[g] + v * v
                        for t in range(1, S):
                            v = rows[p * S + t, sl]
                            mx = jnp.maximum(mx, v)
                            smp = smp + v
                            sq[g] = sq[g] + v * v
                        mh_v[p, sl] = mx - pxv
                        sm[g] = sm[g] + smp
                        cx[g] = cx[g] + pxv * smp
                pltpu.async_copy(mh_v,
                                 mh.at[pl.ds((start + li) * _PC, _PC)],
                                 wsem)
                return tuple(sm) + tuple(sq) + tuple(cx)

            fire(0, rows_a, pxb_a, sga)
            pltpu.async_copy(mh_a, mh.at[pl.ds(start * _PC, _PC)], swa)
            pltpu.async_copy(mh_b, mh.at[pl.ds((start + 1) * _PC, _PC)],
                             swb)

            zeros = jnp.zeros((16,), jnp.float32)
            carry0 = (zeros,) * 12

            def pair(ip, carry):
                ia = 2 * ip
                ib = ia + 1
                fire(ib, rows_b, pxb_b, sgb)
                wait_fire(ia, rows_a, pxb_a, sga)
                carry = compute(ia, rows_a, pxb_a, mh_a, swa, carry)
                fire(ia + 2, rows_a, pxb_a, sga)
                wait_fire(ib, rows_b, pxb_b, sgb)
                carry = compute(ib, rows_b, pxb_b, mh_b, swb, carry)
                return carry

            carry = lax.fori_loop(0, npair, pair, carry0)

            wait_fire(nch - 1, rows_a, pxb_a, sga)
            carry = compute(nch - 1, rows_a, pxb_a, mh_a, swa, carry)
            drain_wb(mh_a, swa)
            drain_wb(mh_b, swb)

            for g in range(4):
                stats_v[pl.ds(sidx * 192 + g * 16, 16)] = carry[g]
                stats_v[pl.ds(sidx * 192 + 64 + g * 16, 16)] = carry[4 + g]
                stats_v[pl.ds(sidx * 192 + 128 + g * 16, 16)] = \
                    carry[8 + g]

        pltpu.sync_copy(stats_v, sth.at[wid])

    return body(t1, t2, t3, i1, i2, i3, x1, x2, x3)


def _bn_affine(s_ref, q_ref, count):
    m = s_ref[...] * (1.0 / count)
    v = q_ref[...] * (1.0 / count) - m * m
    return m, lax.rsqrt(v + EPS)


def _prep_body(vx1, vf1, vx2, vf2, vx3, vf3, pxyz,
               w31, wf1, w32, wf2, w33, wf3, wcat,
               vp1, vp2, vp3, ypos, px1, px2, px3, ys, yq,
               xs1, xq1, xs2, xq2, xs3, xq3):
    f32 = jnp.float32
    vp1[...] = jnp.dot(vx1[...], w31[...], preferred_element_type=f32) + \
               jnp.dot(vf1[...], wf1[...], preferred_element_type=f32)
    vp2[...] = jnp.dot(vx2[...], w32[...], preferred_element_type=f32) + \
               jnp.dot(vf2[...], wf2[...], preferred_element_type=f32)
    vp3[...] = jnp.dot(vx3[...], w33[...], preferred_element_type=f32) + \
               jnp.dot(vf3[...], wf3[...], preferred_element_type=f32)
    p = jnp.dot(pxyz[...], wcat[...], preferred_element_type=f32)
    yp = p[:, 0:128]
    ypos[...] = yp
    a1 = p[:, 128:192]
    a2 = p[:, 192:256]
    a3 = p[:, 256:320]
    px1[...] = a1
    px2[...] = a2
    px3[...] = a3

    @pl.when(pl.program_id(0) == 0)
    def _():
        for r in (ys, yq, xs1, xq1, xs2, xq2, xs3, xq3):
            r[...] = jnp.zeros_like(r)

    ys[...] += jnp.sum(yp, axis=0, keepdims=True)
    yq[...] += jnp.sum(yp * yp, axis=0, keepdims=True)
    xs1[...] += jnp.sum(a1, axis=0, keepdims=True)
    xq1[...] += jnp.sum(a1 * a1, axis=0, keepdims=True)
    xs2[...] += jnp.sum(a2, axis=0, keepdims=True)
    xq2[...] += jnp.sum(a2 * a2, axis=0, keepdims=True)
    xs3[...] += jnp.sum(a3, axis=0, keepdims=True)
    xq3[...] += jnp.sum(a3 * a3, axis=0, keepdims=True)


def _prep_call(vx1, vf1, vx2, vf2, vx3, vf3, pxyz,
               w31, wf1, w32, wf2, w33, wf3, wcat):
    f32 = jnp.float32
    blk = lambda c: pl.BlockSpec((_BPR, c), lambda i: (i, 0))
    full = lambda r, c: pl.BlockSpec((r, c), lambda i: (0, 0))
    return pl.pallas_call(
        _prep_body,
        grid=(N // _BPR,),
        in_specs=[blk(3), blk(32), blk(3), blk(64), blk(3), blk(64),
                  blk(3),
                  full(3, 64), full(32, 64), full(3, 64), full(64, 64),
                  full(3, 64), full(64, 64), full(3, 320)],
        out_specs=[blk(64), blk(64), blk(64), blk(128),
                   blk(64), blk(64), blk(64),
                   full(1, 128), full(1, 128)] + [full(1, 64)] * 6,
        out_shape=[jax.ShapeDtypeStruct((V, 64), f32)] * 3 +
                  [jax.ShapeDtypeStruct((N, 128), f32)] +
                  [jax.ShapeDtypeStruct((N, 64), f32)] * 3 +
                  [jax.ShapeDtypeStruct((1, 128), f32),
                   jax.ShapeDtypeStruct((1, 128), f32)] +
                  [jax.ShapeDtypeStruct((1, 64), f32)] * 6,
        compiler_params=pltpu.CompilerParams(
            dimension_semantics=("arbitrary",)),
    )(vx1, vf1, vx2, vf2, vx3, vf3, pxyz,
      w31, wf1, w32, wf2, w33, wf3, wcat)


def _head1_body(m1, m2, m3, stats, xs1, xq1, xs2, xq2, xs3, xq3,
                wr1, wr2, wr3, yraw, ys, yq):
    cnt = float(N * S)
    st = jnp.sum(stats[...], axis=0, keepdims=True)
    f32 = jnp.float32
    ps = []
    for sidx, (m, xs, xq) in enumerate(((m1, xs1, xq1), (m2, xs2, xq2),
                                        (m3, xs3, xq3))):
        sv = st[:, sidx * 192:sidx * 192 + 64]
        qv = st[:, sidx * 192 + 64:sidx * 192 + 128]
        cv = st[:, sidx * 192 + 128:sidx * 192 + 192]
        hs = sv - float(S) * xs[...]
        hq = qv - 2.0 * cv + float(S) * xq[...]
        mu = hs * (1.0 / cnt)
        var = hq * (1.0 / cnt) - mu * mu
        rs = lax.rsqrt(var + EPS)
        ps.append(jnp.maximum((m[...] - mu) * rs, 0.0))
    y = jnp.dot(ps[0], wr1[...], preferred_element_type=f32) + \
        jnp.dot(ps[1], wr2[...], preferred_element_type=f32) + \
        jnp.dot(ps[2], wr3[...], preferred_element_type=f32)
    yraw[...] = y

    @pl.when(pl.program_id(0) == 0)
    def _():
        ys[...] = jnp.zeros_like(ys)
        yq[...] = jnp.zeros_like(yq)

    ys[...] += jnp.sum(y, axis=0, keepdims=True)
    yq[...] += jnp.sum(y * y, axis=0, keepdims=True)


def _head1_call(m1, m2, m3, stats, pxstats, wr1, wr2, wr3):
    f32 = jnp.float32
    blk64 = pl.BlockSpec((_BP, 64), lambda i: (i, 0))
    st64 = pl.BlockSpec((1, 64), lambda i: (0, 0))
    stw = pl.BlockSpec((_NW, _STW), lambda i: (0, 0))
    w = pl.BlockSpec((64, 128), lambda i: (0, 0))
    return pl.pallas_call(
        _head1_body,
        grid=(_NB,),
        in_specs=[blk64, blk64, blk64, stw] + [st64] * 6 + [w, w, w],
        out_specs=[pl.BlockSpec((_BP, 128), lambda i: (i, 0)),
                   pl.BlockSpec((1, 128), lambda i: (0, 0)),
                   pl.BlockSpec((1, 128), lambda i: (0, 0))],
        out_shape=[jax.ShapeDtypeStruct((N, 128), f32),
                   jax.ShapeDtypeStruct((1, 128), f32),
                   jax.ShapeDtypeStruct((1, 128), f32)],
        compiler_params=pltpu.CompilerParams(
            dimension_semantics=("arbitrary",)),
    )(m1, m2, m3, stats, *pxstats, wr1, wr2, wr3)


def _head2_body(yraw, ypos, rs_, rq_, ps_, pq_, wfg1, wct1,
                z1, z2, s1, q1, s2, q2):
    cnt = float(N)
    mur, rsr = _bn_affine(rs_, rq_, cnt)
    mup, rsp = _bn_affine(ps_, pq_, cnt)
    feat = jnp.maximum((yraw[...] - mur) * rsr + (ypos[...] - mup) * rsp,
                       0.0)
    f32 = jnp.float32
    a = jnp.dot(feat, wfg1[...], preferred_element_type=f32)
    b = jnp.dot(feat, wct1[...], preferred_element_type=f32)
    z1[...] = a
    z2[...] = b

    @pl.when(pl.program_id(0) == 0)
    def _():
        s1[...] = jnp.zeros_like(s1)
        q1[...] = jnp.zeros_like(q1)
        s2[...] = jnp.zeros_like(s2)
        q2[...] = jnp.zeros_like(q2)

    s1[...] += jnp.sum(a, axis=0, keepdims=True)
    q1[...] += jnp.sum(a * a, axis=0, keepdims=True)
    s2[...] += jnp.sum(b, axis=0, keepdims=True)
    q2[...] += jnp.sum(b * b, axis=0, keepdims=True)


def _head2_call(yraw, ypos, rs_, rq_, ps_, pq_, wfg1, wct1):
    f32 = jnp.float32
    blk128 = pl.BlockSpec((_BP, 128), lambda i: (i, 0))
    st = pl.BlockSpec((1, 128), lambda i: (0, 0))
    w = pl.BlockSpec((128, 64), lambda i: (0, 0))
    st64 = pl.BlockSpec((1, 64), lambda i: (0, 0))
    return pl.pallas_call(
        _head2_body,
        grid=(_NB,),
        in_specs=[blk128, blk128, st, st, st, st, w, w],
        out_specs=[pl.BlockSpec((_BP, 64), lambda i: (i, 0))] * 2 +
                  [st64, st64, st64, st64],
        out_shape=[jax.ShapeDtypeStruct((N, 64), f32)] * 2 +
                  [jax.ShapeDtypeStruct((1, 64), f32)] * 4,
        compiler_params=pltpu.CompilerParams(
            dimension_semantics=("arbitrary",)),
    )(yraw, ypos, rs_, rq_, ps_, pq_, wfg1, wct1)


def _head3_body(z1, z2, s1, q1, s2, q2, wfg2, wct2, bcat, out):
    cnt = float(N)
    mu1, rs1 = _bn_affine(s1, q1, cnt)
    mu2, rs2 = _bn_affine(s2, q2, cnt)
    a1 = jnp.maximum((z1[...] - mu1) * rs1, 0.0)
    a2 = jnp.maximum((z2[...] - mu2) * rs2, 0.0)
    f32 = jnp.float32
    out[...] = jnp.dot(a1, wfg2[...], preferred_element_type=f32) + \
               jnp.dot(a2, wct2[...], preferred_element_type=f32) + \
               bcat[...]


def _head3_call(z1, z2, s1, q1, s2, q2, wfg2, wct2, bcat):
    f32 = jnp.float32
    blk64 = pl.BlockSpec((_BP, 64), lambda i: (i, 0))
    st = pl.BlockSpec((1, 64), lambda i: (0, 0))
    w = pl.BlockSpec((64, 6), lambda i: (0, 0))
    return pl.pallas_call(
        _head3_body,
        grid=(_NB,),
        in_specs=[blk64, blk64, st, st, st, st, w, w,
                  pl.BlockSpec((1, 6), lambda i: (0, 0))],
        out_specs=pl.BlockSpec((_BP, 6), lambda i: (i, 0)),
        out_shape=jax.ShapeDtypeStruct((N, 6), f32),
        compiler_params=pltpu.CompilerParams(
            dimension_semantics=("arbitrary",)),
    )(z1, z2, s1, q1, s2, q2, wfg2, wct2, bcat)


def kernel(point_xyz, vx1, vx2, vx3, vf1, vf2, vf3, idx1, idx2, idx3,
           W_s1, W_s2, W_s3, W_raw, W_pos, W_fg1, W_fg2, b_fg,
           W_ct1, W_ct2, b_ct):
    f32 = jnp.float32
    w31, wf1 = W_s1[:3], W_s1[3:]
    w32, wf2 = W_s2[:3], W_s2[3:]
    w33, wf3 = W_s3[:3], W_s3[3:]
    wcat = jnp.concatenate([W_pos, w31, w32, w33], axis=1)
    wr1, wr2, wr3 = W_raw[0:64], W_raw[64:128], W_raw[128:192]
    wfg2p = jnp.concatenate([W_fg2, jnp.zeros((64, 3), f32)], axis=1)
    wct2p = jnp.concatenate([jnp.zeros((64, 3), f32), W_ct2], axis=1)
    bcat = jnp.concatenate([b_fg, b_ct]).reshape(1, 6)

    (vp1, vp2, vp3, ypos, px1, px2, px3, ps_, pq_,
     xs1, xq1, xs2, xq2, xs3, xq3) = _prep_call(
        vx1, vf1, vx2, vf2, vx3, vf3, point_xyz,
        w31, wf1, w32, wf2, w33, wf3, wcat)

    m1, m2, m3, stats = _sc_pool_call(
        vp1, vp2, vp3, idx1.reshape(-1), idx2.reshape(-1),
        idx3.reshape(-1), px1, px2, px3)

    yraw, rs_, rq_ = _head1_call(m1, m2, m3, stats,
                                 (xs1, xq1, xs2, xq2, xs3, xq3),
                                 wr1, wr2, wr3)
    z1, z2, zs1, zq1, zs2, zq2 = _head2_call(yraw, ypos, rs_, rq_,
                                             ps_, pq_, W_fg1, W_ct1)
    return _head3_call(z1, z2, zs1, zq1, zs2, zq2, wfg2p, wct2p, bcat)
